# baseline (speedup 1.0000x reference)
import functools

import jax
import jax.numpy as jnp
from jax import lax
from jax.experimental import pallas as pl
from jax.experimental.pallas import tpu as pltpu
from jax.experimental.pallas import tpu_sc as plsc

C = 1000
N = 16384
D = 1024
NC = 2
NS = 16
L = 16
COLS = D // NS
ROWS_PER_CORE = N // NC
RB = 256
NBLK = ROWS_PER_CORE // RB
NW = NC * NS


def _sc_partials(x, labels, centers):
    mesh = plsc.VectorSubcoreMesh(
        core_axis_name="c", subcore_axis_name="s",
        num_cores=NC, num_subcores=NS)

    @functools.partial(
        pl.kernel,
        out_type=jax.ShapeDtypeStruct((NW, C, L), jnp.float32),
        mesh=mesh,
        scratch_types=[
            pltpu.VMEM((C, COLS), jnp.float32),
            pltpu.VMEM((RB, COLS), jnp.float32),
            pltpu.VMEM((RB,), jnp.int32),
            pltpu.VMEM((C, L), jnp.float32),
        ],
        compiler_params=pltpu.CompilerParams(use_tc_tiling_on_sc=False),
    )
    def k(x_hbm, lbl_hbm, cen_hbm, out_hbm, cbuf, xbuf, lbuf, pacc):
        cid = lax.axis_index("c")
        sid = lax.axis_index("s")
        col0 = sid * COLS
        row0_core = cid * ROWS_PER_CORE

        pltpu.sync_copy(cen_hbm.at[:, pl.ds(col0, COLS)], cbuf)

        zero = jnp.zeros((L,), jnp.float32)

        def zero_body(i, carry):
            pacc[i] = zero
            return carry

        lax.fori_loop(0, C, zero_body, 0)

        def blk_body(b, carry):
            r0 = row0_core + b * RB
            pltpu.sync_copy(x_hbm.at[pl.ds(r0, RB), pl.ds(col0, COLS)], xbuf)
            pltpu.sync_copy(lbl_hbm.at[pl.ds(r0, RB)], lbuf)

            def grp_body(g, carry2):
                lblv = lbuf[pl.ds(g * L, L)]
                for j in range(L):
                    lab = lblv[j]
                    r = g * L + j
                    acc = jnp.zeros((L,), jnp.float32)
                    for kk in range(COLS // L):
                        xv = xbuf[r, pl.ds(kk * L, L)]
                        cv = cbuf[lab, pl.ds(kk * L, L)]
                        df = xv - cv
                        acc = acc + df * df
                    plsc.addupdate(pacc.at[lab], acc)
                return carry2

            lax.fori_loop(0, RB // L, grp_body, 0)
            return carry

        lax.fori_loop(0, NBLK, blk_body, 0)

        wid = sid * NC + cid
        pltpu.sync_copy(pacc, out_hbm.at[wid])

    return k(x, labels, centers)


def _finish(partials):
    def body(p_ref, o_ref):
        s = jnp.sum(p_ref[...], axis=(0, 2))
        mask = s > 0.0
        norms = jnp.where(mask, jnp.sqrt(jnp.where(mask, s, 1.0)), 0.0)
        o_ref[0, 0] = jnp.sum(norms) / C

    return pl.pallas_call(
        body,
        out_shape=jax.ShapeDtypeStruct((1, 1), jnp.float32),
        in_specs=[pl.BlockSpec(memory_space=pltpu.VMEM)],
        out_specs=pl.BlockSpec(memory_space=pltpu.SMEM),
    )(partials)[0, 0]


def kernel(x, labels, centers):
    partials = _sc_partials(x, labels, centers)
    return _finish(partials)


# speedup vs baseline: 1.2043x; 1.2043x over previous
import functools

import jax
import jax.numpy as jnp
from jax import lax
from jax.experimental import pallas as pl
from jax.experimental.pallas import tpu as pltpu
from jax.experimental.pallas import tpu_sc as plsc

C = 1000
CP = 1024
N = 16384
D = 1024
NC = 2
NS = 16
L = 16
COLS = D // NS
ROWS_PER_CORE = N // NC
RB = 128
NBLK = ROWS_PER_CORE // RB
AW = 32


def _sc_stats(x, labels):
    mesh = plsc.VectorSubcoreMesh(
        core_axis_name="c", subcore_axis_name="s",
        num_cores=NC, num_subcores=NS)

    @functools.partial(
        pl.kernel,
        out_type=(
            jax.ShapeDtypeStruct((NC, CP, NS, COLS), jnp.float32),
            jax.ShapeDtypeStruct((NC, CP, NS, AW), jnp.float32),
        ),
        mesh=mesh,
        scratch_types=[
            pltpu.VMEM((CP, COLS), jnp.float32),
            pltpu.VMEM((CP, AW), jnp.float32),
            pltpu.VMEM((RB, COLS), jnp.float32),
            pltpu.VMEM((RB, COLS), jnp.float32),
            pltpu.VMEM((RB,), jnp.int32),
            pltpu.VMEM((RB,), jnp.int32),
            pltpu.SemaphoreType.DMA,
            pltpu.SemaphoreType.DMA,
        ],
        compiler_params=pltpu.CompilerParams(use_tc_tiling_on_sc=False),
    )
    def k(x_hbm, lbl_hbm, m_hbm, aux_hbm,
          macc, aacc, xb0, xb1, lb0, lb1, si0, si1):
        cid = lax.axis_index("c")
        sid = lax.axis_index("s")
        row0 = cid * ROWS_PER_CORE
        col0 = sid * COLS

        zero = jnp.zeros((L,), jnp.float32)
        iota = lax.iota(jnp.int32, L)
        onev = jnp.where(iota == 0, 1.0, 0.0).astype(jnp.float32)

        def zero_body(i, carry):
            for j in range(COLS // L):
                macc[i, pl.ds(j * L, L)] = zero
            for j in range(AW // L):
                aacc[i, pl.ds(j * L, L)] = zero
            return carry

        lax.fori_loop(0, CP, zero_body, 0)

        xbs, lbs, sis = (xb0, xb1), (lb0, lb1), (si0, si1)

        def issue_in(b, p):
            r0 = row0 + (b % NBLK) * RB
            pltpu.async_copy(
                x_hbm.at[pl.ds(r0, RB), pl.ds(col0, COLS)], xbs[p], sis[p])
            pltpu.async_copy(lbl_hbm.at[pl.ds(r0, RB)], lbs[p], sis[p])

        def wait_in(p):
            pltpu.make_async_copy(
                x_hbm.at[pl.ds(row0, RB), pl.ds(col0, COLS)],
                xbs[p], sis[p]).wait()
            pltpu.make_async_copy(
                lbl_hbm.at[pl.ds(row0, RB)], lbs[p], sis[p]).wait()

        issue_in(0, 0)
        issue_in(1, 1)

        def pair_body(i, carry):
            for t in range(2):
                b = 2 * i + t
                wait_in(t)
                xb, lb = xbs[t], lbs[t]

                def grp_body(g, carry2):
                    lblv = lb[pl.ds(g * L, L)]
                    for j in range(L):
                        lab = lblv[j]
                        r = g * L + j
                        v0 = xb[r, pl.ds(0, L)]
                        v1 = xb[r, pl.ds(L, L)]
                        v2 = xb[r, pl.ds(2 * L, L)]
                        v3 = xb[r, pl.ds(3 * L, L)]
                        plsc.addupdate(macc.at[lab, pl.ds(0, L)], v0)
                        plsc.addupdate(macc.at[lab, pl.ds(L, L)], v1)
                        plsc.addupdate(macc.at[lab, pl.ds(2 * L, L)], v2)
                        plsc.addupdate(macc.at[lab, pl.ds(3 * L, L)], v3)
                        sq = ((v0 * v0 + v1 * v1) + (v2 * v2 + v3 * v3))
                        plsc.addupdate(aacc.at[lab, pl.ds(0, L)], sq)
                        plsc.addupdate(aacc.at[lab, pl.ds(L, L)], onev)
                    return carry2

                lax.fori_loop(0, RB // L, grp_body, 0)
                issue_in(b + 2, t)
            return carry

        lax.fori_loop(0, NBLK // 2, pair_body, 0)

        wait_in(0)
        wait_in(1)

        pltpu.sync_copy(macc, m_hbm.at[cid, :, sid])
        pltpu.sync_copy(aacc, aux_hbm.at[cid, :, sid])

    return k(x, labels)


CB = 128


def _finish(m_parts, aux_parts, centers_padded):
    def body(m_ref, a_ref, c_ref, o_ref):
        g = pl.program_id(0)
        cen = c_ref[...]
        m = (m_ref[0] + m_ref[1]).reshape(CB, D)
        aux = a_ref[0] + a_ref[1]
        cross = jnp.sum(m * cen, axis=1)
        s = jnp.sum(aux[:, :, 0:L], axis=(1, 2))
        cnt = jnp.sum(aux[:, :, L], axis=1) / NS
        cn2 = jnp.sum(cen * cen, axis=1)
        pc = s - 2.0 * cross + cnt * cn2
        mask = pc > 0.0
        norms = jnp.where(mask, jnp.sqrt(jnp.where(mask, pc, 1.0)), 0.0)
        part = jnp.sum(norms) / C

        @pl.when(g == 0)
        def _():
            o_ref[0, 0] = 0.0

        o_ref[0, 0] += part

    return pl.pallas_call(
        body,
        grid=(CP // CB,),
        out_shape=jax.ShapeDtypeStruct((1, 1), jnp.float32),
        in_specs=[
            pl.BlockSpec((NC, CB, NS, COLS), lambda g: (0, g, 0, 0)),
            pl.BlockSpec((NC, CB, NS, AW), lambda g: (0, g, 0, 0)),
            pl.BlockSpec((CB, D), lambda g: (g, 0)),
        ],
        out_specs=pl.BlockSpec((1, 1), lambda g: (0, 0),
                               memory_space=pltpu.SMEM),
    )(m_parts, aux_parts, centers_padded)[0, 0]


def kernel(x, labels, centers):
    m_parts, aux_parts = _sc_stats(x, labels)
    centers_padded = jnp.pad(centers, ((0, CP - C), (0, 0)))
    return _finish(m_parts, aux_parts, centers_padded)


# speedup vs baseline: 1.2379x; 1.0279x over previous
import functools

import jax
import jax.numpy as jnp
from jax import lax
from jax.experimental import pallas as pl
from jax.experimental.pallas import tpu as pltpu
from jax.experimental.pallas import tpu_sc as plsc

C = 1000
CP = 1024
N = 16384
D = 1024
NC = 2
NS = 16
L = 16
COLS = D // NS
ROWS_PER_CORE = N // NC
RB = 128
NBLK = ROWS_PER_CORE // RB
AW = 32


def _sc_stats(x, labels):
    mesh = plsc.VectorSubcoreMesh(
        core_axis_name="c", subcore_axis_name="s",
        num_cores=NC, num_subcores=NS)

    @functools.partial(
        pl.kernel,
        out_type=(
            jax.ShapeDtypeStruct((NC, CP, NS, COLS), jnp.float32),
            jax.ShapeDtypeStruct((NC, CP, NS, AW), jnp.float32),
        ),
        mesh=mesh,
        scratch_types=[
            pltpu.VMEM((CP, COLS), jnp.float32),
            pltpu.VMEM((CP, AW), jnp.float32),
            pltpu.VMEM((RB, COLS), jnp.float32),
            pltpu.VMEM((RB, COLS), jnp.float32),
            pltpu.VMEM((RB,), jnp.int32),
            pltpu.VMEM((RB,), jnp.int32),
            pltpu.SemaphoreType.DMA,
            pltpu.SemaphoreType.DMA,
        ],
        compiler_params=pltpu.CompilerParams(use_tc_tiling_on_sc=False),
    )
    def k(x_hbm, lbl_hbm, m_hbm, aux_hbm,
          macc, aacc, xb0, xb1, lb0, lb1, si0, si1):
        cid = lax.axis_index("c")
        sid = lax.axis_index("s")
        row0 = cid * ROWS_PER_CORE
        col0 = sid * COLS

        zero = jnp.zeros((L,), jnp.float32)
        iota = lax.iota(jnp.int32, L)
        onev = jnp.where(iota == 0, 1.0, 0.0).astype(jnp.float32)

        def zero_body(i, carry):
            for j in range(COLS // L):
                macc[i, pl.ds(j * L, L)] = zero
            for j in range(AW // L):
                aacc[i, pl.ds(j * L, L)] = zero
            return carry

        lax.fori_loop(0, CP, zero_body, 0)

        xbs, lbs, sis = (xb0, xb1), (lb0, lb1), (si0, si1)

        def issue_in(b, p):
            r0 = row0 + (b % NBLK) * RB
            pltpu.async_copy(
                x_hbm.at[pl.ds(r0, RB), pl.ds(col0, COLS)], xbs[p], sis[p])
            pltpu.async_copy(lbl_hbm.at[pl.ds(r0, RB)], lbs[p], sis[p])

        def wait_in(p):
            pltpu.make_async_copy(
                x_hbm.at[pl.ds(row0, RB), pl.ds(col0, COLS)],
                xbs[p], sis[p]).wait()
            pltpu.make_async_copy(
                lbl_hbm.at[pl.ds(row0, RB)], lbs[p], sis[p]).wait()

        issue_in(0, 0)
        issue_in(1, 1)

        def pair_body(i, carry):
            for t in range(2):
                b = 2 * i + t
                wait_in(t)
                xb, lb = xbs[t], lbs[t]

                def grp_body(g, carry2):
                    lblv = lb[pl.ds(g * L, L)]
                    for j in range(L):
                        lab = lblv[j]
                        r = g * L + j
                        v0 = xb[r, pl.ds(0, L)]
                        v1 = xb[r, pl.ds(L, L)]
                        v2 = xb[r, pl.ds(2 * L, L)]
                        v3 = xb[r, pl.ds(3 * L, L)]
                        plsc.addupdate(macc.at[lab, pl.ds(0, L)], v0)
                        plsc.addupdate(macc.at[lab, pl.ds(L, L)], v1)
                        plsc.addupdate(macc.at[lab, pl.ds(2 * L, L)], v2)
                        plsc.addupdate(macc.at[lab, pl.ds(3 * L, L)], v3)
                        sq = ((v0 * v0 + v1 * v1) + (v2 * v2 + v3 * v3))
                        plsc.addupdate(aacc.at[lab, pl.ds(0, L)], sq)
                        plsc.addupdate(aacc.at[lab, pl.ds(L, L)], onev)
                    return carry2

                lax.fori_loop(0, RB // L, grp_body, 0)
                issue_in(b + 2, t)
            return carry

        lax.fori_loop(0, NBLK // 2, pair_body, 0)

        wait_in(0)
        wait_in(1)

        pltpu.sync_copy(macc, m_hbm.at[cid, :, sid])
        pltpu.sync_copy(aacc, aux_hbm.at[cid, :, sid])

    return k(x, labels)


CB = 200


def _finish(m_parts, aux_parts, centers):
    def body(m_ref, a_ref, c_ref, o_ref):
        g = pl.program_id(0)
        cen = c_ref[...]
        m = (m_ref[0] + m_ref[1]).reshape(CB, D)
        aux = a_ref[0] + a_ref[1]
        cross = jnp.sum(m * cen, axis=1)
        s = jnp.sum(aux[:, :, 0:L], axis=(1, 2))
        cnt = jnp.sum(aux[:, :, L], axis=1) / NS
        cn2 = jnp.sum(cen * cen, axis=1)
        pc = s - 2.0 * cross + cnt * cn2
        mask = pc > 0.0
        norms = jnp.where(mask, jnp.sqrt(jnp.where(mask, pc, 1.0)), 0.0)
        part = jnp.sum(norms) / C

        @pl.when(g == 0)
        def _():
            o_ref[0, 0] = 0.0

        o_ref[0, 0] += part

    return pl.pallas_call(
        body,
        grid=(C // CB,),
        out_shape=jax.ShapeDtypeStruct((1, 1), jnp.float32),
        in_specs=[
            pl.BlockSpec((NC, CB, NS, COLS), lambda g: (0, g, 0, 0)),
            pl.BlockSpec((NC, CB, NS, AW), lambda g: (0, g, 0, 0)),
            pl.BlockSpec((CB, D), lambda g: (g, 0)),
        ],
        out_specs=pl.BlockSpec((1, 1), lambda g: (0, 0),
                               memory_space=pltpu.SMEM),
    )(m_parts, aux_parts, centers)[0, 0]


def kernel(x, labels, centers):
    m_parts, aux_parts = _sc_stats(x, labels)
    return _finish(m_parts, aux_parts, centers)


# speedup vs baseline: 1.4512x; 1.1723x over previous
import functools

import jax
import jax.numpy as jnp
from jax import lax
from jax.experimental import pallas as pl
from jax.experimental.pallas import tpu as pltpu
from jax.experimental.pallas import tpu_sc as plsc

C = 1000
CP = 1024
N = 16384
D = 1024
NC = 2
NS = 16
L = 16
COLS = D // NS
F_TC = 8192
ROWS_PER_CORE = (N - F_TC) // NC
RB = 128
NBLK = ROWS_PER_CORE // RB
AW = 32
RBT = 512


def _sc_stats(x, labels):
    mesh = plsc.VectorSubcoreMesh(
        core_axis_name="c", subcore_axis_name="s",
        num_cores=NC, num_subcores=NS)

    @functools.partial(
        pl.kernel,
        out_type=(
            jax.ShapeDtypeStruct((NC, CP, NS, COLS), jnp.float32),
            jax.ShapeDtypeStruct((NC, CP, NS, AW), jnp.float32),
        ),
        mesh=mesh,
        scratch_types=[
            pltpu.VMEM((CP, COLS), jnp.float32),
            pltpu.VMEM((CP, AW), jnp.float32),
            pltpu.VMEM((RB, COLS), jnp.float32),
            pltpu.VMEM((RB, COLS), jnp.float32),
            pltpu.VMEM((RB,), jnp.int32),
            pltpu.VMEM((RB,), jnp.int32),
            pltpu.SemaphoreType.DMA,
            pltpu.SemaphoreType.DMA,
        ],
        compiler_params=pltpu.CompilerParams(use_tc_tiling_on_sc=False),
    )
    def k(x_hbm, lbl_hbm, m_hbm, aux_hbm,
          macc, aacc, xb0, xb1, lb0, lb1, si0, si1):
        cid = lax.axis_index("c")
        sid = lax.axis_index("s")
        row0 = F_TC + cid * ROWS_PER_CORE
        col0 = sid * COLS

        zero = jnp.zeros((L,), jnp.float32)
        iota = lax.iota(jnp.int32, L)
        onev = jnp.where(iota == 0, 1.0, 0.0).astype(jnp.float32)

        def zero_body(i, carry):
            for j in range(COLS // L):
                macc[i, pl.ds(j * L, L)] = zero
            for j in range(AW // L):
                aacc[i, pl.ds(j * L, L)] = zero
            return carry

        lax.fori_loop(0, CP, zero_body, 0)

        xbs, lbs, sis = (xb0, xb1), (lb0, lb1), (si0, si1)

        def issue_in(b, p):
            r0 = row0 + (b % NBLK) * RB
            pltpu.async_copy(
                x_hbm.at[pl.ds(r0, RB), pl.ds(col0, COLS)], xbs[p], sis[p])
            pltpu.async_copy(lbl_hbm.at[pl.ds(r0, RB)], lbs[p], sis[p])

        def wait_in(p):
            pltpu.make_async_copy(
                x_hbm.at[pl.ds(row0, RB), pl.ds(col0, COLS)],
                xbs[p], sis[p]).wait()
            pltpu.make_async_copy(
                lbl_hbm.at[pl.ds(row0, RB)], lbs[p], sis[p]).wait()

        issue_in(0, 0)
        issue_in(1, 1)

        def pair_body(i, carry):
            for t in range(2):
                b = 2 * i + t
                wait_in(t)
                xb, lb = xbs[t], lbs[t]

                def grp_body(g, carry2):
                    lblv = lb[pl.ds(g * L, L)]
                    for j in range(L):
                        lab = lblv[j]
                        r = g * L + j
                        v0 = xb[r, pl.ds(0, L)]
                        v1 = xb[r, pl.ds(L, L)]
                        v2 = xb[r, pl.ds(2 * L, L)]
                        v3 = xb[r, pl.ds(3 * L, L)]
                        plsc.addupdate(macc.at[lab, pl.ds(0, L)], v0)
                        plsc.addupdate(macc.at[lab, pl.ds(L, L)], v1)
                        plsc.addupdate(macc.at[lab, pl.ds(2 * L, L)], v2)
                        plsc.addupdate(macc.at[lab, pl.ds(3 * L, L)], v3)
                        sq = ((v0 * v0 + v1 * v1) + (v2 * v2 + v3 * v3))
                        plsc.addupdate(aacc.at[lab, pl.ds(0, L)], sq)
                        plsc.addupdate(aacc.at[lab, pl.ds(L, L)], onev)
                    return carry2

                lax.fori_loop(0, RB // L, grp_body, 0)
                issue_in(b + 2, t)
            return carry

        lax.fori_loop(0, NBLK // 2, pair_body, 0)

        wait_in(0)
        wait_in(1)

        pltpu.sync_copy(macc, m_hbm.at[cid, :, sid])
        pltpu.sync_copy(aacc, aux_hbm.at[cid, :, sid])

    return k(x, labels)


def _tc_stats(x_tc, labels3):
    def body(x_ref, l_ref, m_ref, a_ref):
        g = pl.program_id(0)

        @pl.when(g == 0)
        def _():
            m_ref[...] = jnp.zeros_like(m_ref)
            a_ref[...] = jnp.zeros_like(a_ref)

        xb = x_ref[...]
        lab = l_ref[0, 0, :]
        cls = jax.lax.broadcasted_iota(jnp.int32, (CP, RBT), 0)
        oh = (lab[None, :] == cls)
        ohb = oh.astype(jnp.bfloat16)
        xb16 = xb.astype(jnp.bfloat16)
        m_ref[...] += jax.lax.dot_general(
            ohb, xb16, (((1,), (0,)), ((), ())),
            preferred_element_type=jnp.float32)
        x2 = jnp.sum(xb * xb, axis=1)
        rhs = jnp.stack([x2, jnp.ones_like(x2)], axis=1)
        ohf = oh.astype(jnp.float32)
        sc2 = jax.lax.dot_general(
            ohf, rhs, (((1,), (0,)), ((), ())),
            preferred_element_type=jnp.float32)
        a_ref[:, 0:2] += sc2

    return pl.pallas_call(
        body,
        grid=(F_TC // RBT,),
        out_shape=(
            jax.ShapeDtypeStruct((CP, D), jnp.float32),
            jax.ShapeDtypeStruct((CP, 128), jnp.float32),
        ),
        in_specs=[
            pl.BlockSpec((RBT, D), lambda g: (g, 0)),
            pl.BlockSpec((1, 1, RBT), lambda g: (g, 0, 0)),
        ],
        out_specs=(
            pl.BlockSpec((CP, D), lambda g: (0, 0)),
            pl.BlockSpec((CP, 128), lambda g: (0, 0)),
        ),
    )(x_tc, labels3)


CB = 200


def _finish(m_parts, aux_parts, m_tc, aux_tc, centers):
    def body(m_ref, a_ref, mt_ref, at_ref, c_ref, o_ref):
        g = pl.program_id(0)
        cen = c_ref[...]
        m = (m_ref[0] + m_ref[1]).reshape(CB, D) + mt_ref[...]
        aux = a_ref[0] + a_ref[1]
        cross = jnp.sum(m * cen, axis=1)
        s = jnp.sum(aux[:, :, 0:L], axis=(1, 2)) + at_ref[:, 0]
        cnt = jnp.sum(aux[:, :, L], axis=1) / NS + at_ref[:, 1]
        cn2 = jnp.sum(cen * cen, axis=1)
        pc = s - 2.0 * cross + cnt * cn2
        mask = pc > 0.0
        norms = jnp.where(mask, jnp.sqrt(jnp.where(mask, pc, 1.0)), 0.0)
        part = jnp.sum(norms) / C

        @pl.when(g == 0)
        def _():
            o_ref[0, 0] = 0.0

        o_ref[0, 0] += part

    return pl.pallas_call(
        body,
        grid=(C // CB,),
        out_shape=jax.ShapeDtypeStruct((1, 1), jnp.float32),
        in_specs=[
            pl.BlockSpec((NC, CB, NS, COLS), lambda g: (0, g, 0, 0)),
            pl.BlockSpec((NC, CB, NS, AW), lambda g: (0, g, 0, 0)),
            pl.BlockSpec((CB, D), lambda g: (g, 0)),
            pl.BlockSpec((CB, 128), lambda g: (g, 0)),
            pl.BlockSpec((CB, D), lambda g: (g, 0)),
        ],
        out_specs=pl.BlockSpec((1, 1), lambda g: (0, 0),
                               memory_space=pltpu.SMEM),
    )(m_parts, aux_parts, m_tc, aux_tc, centers)[0, 0]


def kernel(x, labels, centers):
    m_parts, aux_parts = _sc_stats(x, labels)
    labels3 = labels[:F_TC].reshape(F_TC // RBT, 1, RBT)
    m_tc, aux_tc = _tc_stats(x[:F_TC], labels3)
    return _finish(m_parts, aux_parts, m_tc, aux_tc, centers)


# speedup vs baseline: 1.5410x; 1.0619x over previous
import functools

import jax
import jax.numpy as jnp
from jax import lax
from jax.experimental import pallas as pl
from jax.experimental.pallas import tpu as pltpu
from jax.experimental.pallas import tpu_sc as plsc

C = 1000
CP = 1024
N = 16384
D = 1024
NC = 2
NS = 16
L = 16
COLS = D // NS
F_TC = 12288
ROWS_PER_CORE = (N - F_TC) // NC
RB = 128
NBLK = ROWS_PER_CORE // RB
AW = 32
RBT = 512


def _sc_stats(x, labels):
    mesh = plsc.VectorSubcoreMesh(
        core_axis_name="c", subcore_axis_name="s",
        num_cores=NC, num_subcores=NS)

    @functools.partial(
        pl.kernel,
        out_type=(
            jax.ShapeDtypeStruct((NC, CP, NS, COLS), jnp.float32),
            jax.ShapeDtypeStruct((NC, CP, NS, AW), jnp.float32),
        ),
        mesh=mesh,
        scratch_types=[
            pltpu.VMEM((CP, COLS), jnp.float32),
            pltpu.VMEM((CP, AW), jnp.float32),
            pltpu.VMEM((RB, COLS), jnp.float32),
            pltpu.VMEM((RB, COLS), jnp.float32),
            pltpu.VMEM((RB,), jnp.int32),
            pltpu.VMEM((RB,), jnp.int32),
            pltpu.SemaphoreType.DMA,
            pltpu.SemaphoreType.DMA,
        ],
        compiler_params=pltpu.CompilerParams(use_tc_tiling_on_sc=False),
    )
    def k(x_hbm, lbl_hbm, m_hbm, aux_hbm,
          macc, aacc, xb0, xb1, lb0, lb1, si0, si1):
        cid = lax.axis_index("c")
        sid = lax.axis_index("s")
        row0 = cid * ROWS_PER_CORE
        col0 = sid * COLS

        zero = jnp.zeros((L,), jnp.float32)
        iota = lax.iota(jnp.int32, L)
        onev = jnp.where(iota == 0, 1.0, 0.0).astype(jnp.float32)

        def zero_body(i, carry):
            for j in range(COLS // L):
                macc[i, pl.ds(j * L, L)] = zero
            for j in range(AW // L):
                aacc[i, pl.ds(j * L, L)] = zero
            return carry

        lax.fori_loop(0, CP, zero_body, 0)

        xbs, lbs, sis = (xb0, xb1), (lb0, lb1), (si0, si1)

        def issue_in(b, p):
            r0 = row0 + (b % NBLK) * RB
            pltpu.async_copy(
                x_hbm.at[pl.ds(r0, RB), pl.ds(col0, COLS)], xbs[p], sis[p])
            pltpu.async_copy(lbl_hbm.at[pl.ds(r0, RB)], lbs[p], sis[p])

        def wait_in(p):
            pltpu.make_async_copy(
                x_hbm.at[pl.ds(row0, RB), pl.ds(col0, COLS)],
                xbs[p], sis[p]).wait()
            pltpu.make_async_copy(
                lbl_hbm.at[pl.ds(row0, RB)], lbs[p], sis[p]).wait()

        issue_in(0, 0)
        issue_in(1, 1)

        def pair_body(i, carry):
            for t in range(2):
                b = 2 * i + t
                wait_in(t)
                xb, lb = xbs[t], lbs[t]

                def grp_body(g, carry2):
                    lblv = lb[pl.ds(g * L, L)]
                    for j in range(L):
                        lab = lblv[j]
                        r = g * L + j
                        v0 = xb[r, pl.ds(0, L)]
                        v1 = xb[r, pl.ds(L, L)]
                        v2 = xb[r, pl.ds(2 * L, L)]
                        v3 = xb[r, pl.ds(3 * L, L)]
                        plsc.addupdate(macc.at[lab, pl.ds(0, L)], v0)
                        plsc.addupdate(macc.at[lab, pl.ds(L, L)], v1)
                        plsc.addupdate(macc.at[lab, pl.ds(2 * L, L)], v2)
                        plsc.addupdate(macc.at[lab, pl.ds(3 * L, L)], v3)
                        sq = ((v0 * v0 + v1 * v1) + (v2 * v2 + v3 * v3))
                        plsc.addupdate(aacc.at[lab, pl.ds(0, L)], sq)
                        plsc.addupdate(aacc.at[lab, pl.ds(L, L)], onev)
                    return carry2

                lax.fori_loop(0, RB // L, grp_body, 0)
                issue_in(b + 2, t)
            return carry

        lax.fori_loop(0, NBLK // 2, pair_body, 0)

        wait_in(0)
        wait_in(1)

        pltpu.sync_copy(macc, m_hbm.at[cid, :, sid])
        pltpu.sync_copy(aacc, aux_hbm.at[cid, :, sid])

    return k(x, labels)


def _tc_stats(x_tc, labels3):
    def body(x_ref, l_ref, m_ref, a_ref):
        g = pl.program_id(0)

        @pl.when(g == 0)
        def _():
            m_ref[...] = jnp.zeros_like(m_ref)
            a_ref[...] = jnp.zeros_like(a_ref)

        xb = x_ref[...]
        lab = l_ref[0, 0, :]
        cls = jax.lax.broadcasted_iota(jnp.int32, (CP, RBT), 0)
        oh = (lab[None, :] == cls)
        ohb = oh.astype(jnp.bfloat16)
        xb16 = xb.astype(jnp.bfloat16)
        m_ref[...] += jax.lax.dot_general(
            ohb, xb16, (((1,), (0,)), ((), ())),
            preferred_element_type=jnp.float32)
        x2 = jnp.sum(xb * xb, axis=1)
        rhs = jnp.stack([x2, jnp.ones_like(x2)], axis=1)
        ohf = oh.astype(jnp.float32)
        sc2 = jax.lax.dot_general(
            ohf, rhs, (((1,), (0,)), ((), ())),
            preferred_element_type=jnp.float32)
        a_ref[:, 0:2] += sc2

    return pl.pallas_call(
        body,
        grid=(F_TC // RBT,),
        out_shape=(
            jax.ShapeDtypeStruct((CP, D), jnp.float32),
            jax.ShapeDtypeStruct((CP, 128), jnp.float32),
        ),
        in_specs=[
            pl.BlockSpec((RBT, D), lambda g: (g, 0)),
            pl.BlockSpec((1, 1, RBT), lambda g: (g, 0, 0)),
        ],
        out_specs=(
            pl.BlockSpec((CP, D), lambda g: (0, 0)),
            pl.BlockSpec((CP, 128), lambda g: (0, 0)),
        ),
    )(x_tc, labels3)


CB = 200


def _finish(m_parts, aux_parts, m_tc, aux_tc, centers):
    def body(m_ref, a_ref, mt_ref, at_ref, c_ref, o_ref):
        g = pl.program_id(0)
        cen = c_ref[...]
        m = (m_ref[0] + m_ref[1]).reshape(CB, D) + mt_ref[...]
        aux = a_ref[0] + a_ref[1]
        cross = jnp.sum(m * cen, axis=1)
        s = jnp.sum(aux[:, :, 0:L], axis=(1, 2)) + at_ref[:, 0]
        cnt = jnp.sum(aux[:, :, L], axis=1) / NS + at_ref[:, 1]
        cn2 = jnp.sum(cen * cen, axis=1)
        pc = s - 2.0 * cross + cnt * cn2
        mask = pc > 0.0
        norms = jnp.where(mask, jnp.sqrt(jnp.where(mask, pc, 1.0)), 0.0)
        part = jnp.sum(norms) / C

        @pl.when(g == 0)
        def _():
            o_ref[0, 0] = 0.0

        o_ref[0, 0] += part

    return pl.pallas_call(
        body,
        grid=(C // CB,),
        out_shape=jax.ShapeDtypeStruct((1, 1), jnp.float32),
        in_specs=[
            pl.BlockSpec((NC, CB, NS, COLS), lambda g: (0, g, 0, 0)),
            pl.BlockSpec((NC, CB, NS, AW), lambda g: (0, g, 0, 0)),
            pl.BlockSpec((CB, D), lambda g: (g, 0)),
            pl.BlockSpec((CB, 128), lambda g: (g, 0)),
            pl.BlockSpec((CB, D), lambda g: (g, 0)),
        ],
        out_specs=pl.BlockSpec((1, 1), lambda g: (0, 0),
                               memory_space=pltpu.SMEM),
    )(m_parts, aux_parts, m_tc, aux_tc, centers)[0, 0]


def kernel(x, labels, centers):
    m_parts, aux_parts = _sc_stats(x[F_TC:], labels[F_TC:])
    labels3 = labels[:F_TC].reshape(F_TC // RBT, 1, RBT)
    m_tc, aux_tc = _tc_stats(x[:F_TC], labels3)
    return _finish(m_parts, aux_parts, m_tc, aux_tc, centers)


# speedup vs baseline: 1.9344x; 1.2553x over previous
import functools

import jax
import jax.numpy as jnp
from jax import lax
from jax.experimental import pallas as pl
from jax.experimental.pallas import tpu as pltpu
from jax.experimental.pallas import tpu_sc as plsc

C = 1000
CP = 1024
N = 16384
D = 1024
NC = 2
NS = 16
L = 16
COLS = D // NS
F_TC = 12288
ROWS_PER_CORE = (N - F_TC) // NC
RB = 128
NBLK = ROWS_PER_CORE // RB
AW = 32
RBT = 512


def _sc_stats(x, labels):
    mesh = plsc.VectorSubcoreMesh(
        core_axis_name="c", subcore_axis_name="s",
        num_cores=NC, num_subcores=NS)

    @functools.partial(
        pl.kernel,
        out_type=(
            jax.ShapeDtypeStruct((NC, CP, NS, COLS), jnp.float32),
            jax.ShapeDtypeStruct((NC, CP, NS, AW), jnp.float32),
        ),
        mesh=mesh,
        scratch_types=[
            pltpu.VMEM((CP, COLS), jnp.float32),
            pltpu.VMEM((CP, AW), jnp.float32),
            pltpu.VMEM((RB, COLS), jnp.float32),
            pltpu.VMEM((RB, COLS), jnp.float32),
            pltpu.VMEM((RB,), jnp.int32),
            pltpu.VMEM((RB,), jnp.int32),
            pltpu.SemaphoreType.DMA,
            pltpu.SemaphoreType.DMA,
        ],
        compiler_params=pltpu.CompilerParams(use_tc_tiling_on_sc=False),
    )
    def k(x_hbm, lbl_hbm, m_hbm, aux_hbm,
          macc, aacc, xb0, xb1, lb0, lb1, si0, si1):
        cid = lax.axis_index("c")
        sid = lax.axis_index("s")
        row0 = cid * ROWS_PER_CORE
        col0 = sid * COLS

        zero = jnp.zeros((L,), jnp.float32)
        iota = lax.iota(jnp.int32, L)
        onev = jnp.where(iota == 0, 1.0, 0.0).astype(jnp.float32)

        def zero_body(i, carry):
            for j in range(COLS // L):
                macc[i, pl.ds(j * L, L)] = zero
            for j in range(AW // L):
                aacc[i, pl.ds(j * L, L)] = zero
            return carry

        lax.fori_loop(0, CP, zero_body, 0)

        xbs, lbs, sis = (xb0, xb1), (lb0, lb1), (si0, si1)

        def issue_in(b, p):
            r0 = row0 + (b % NBLK) * RB
            pltpu.async_copy(
                x_hbm.at[pl.ds(r0, RB), pl.ds(col0, COLS)], xbs[p], sis[p])
            pltpu.async_copy(lbl_hbm.at[pl.ds(r0, RB)], lbs[p], sis[p])

        def wait_in(p):
            pltpu.make_async_copy(
                x_hbm.at[pl.ds(row0, RB), pl.ds(col0, COLS)],
                xbs[p], sis[p]).wait()
            pltpu.make_async_copy(
                lbl_hbm.at[pl.ds(row0, RB)], lbs[p], sis[p]).wait()

        issue_in(0, 0)
        issue_in(1, 1)

        def pair_body(i, carry):
            for t in range(2):
                b = 2 * i + t
                wait_in(t)
                xb, lb = xbs[t], lbs[t]

                def grp_body(g, carry2):
                    lblv = lb[pl.ds(g * L, L)]
                    for j in range(L):
                        lab = lblv[j]
                        r = g * L + j
                        v0 = xb[r, pl.ds(0, L)]
                        v1 = xb[r, pl.ds(L, L)]
                        v2 = xb[r, pl.ds(2 * L, L)]
                        v3 = xb[r, pl.ds(3 * L, L)]
                        plsc.addupdate(macc.at[lab, pl.ds(0, L)], v0)
                        plsc.addupdate(macc.at[lab, pl.ds(L, L)], v1)
                        plsc.addupdate(macc.at[lab, pl.ds(2 * L, L)], v2)
                        plsc.addupdate(macc.at[lab, pl.ds(3 * L, L)], v3)
                        sq = ((v0 * v0 + v1 * v1) + (v2 * v2 + v3 * v3))
                        plsc.addupdate(aacc.at[lab, pl.ds(0, L)], sq)
                        plsc.addupdate(aacc.at[lab, pl.ds(L, L)], onev)
                    return carry2

                lax.fori_loop(0, RB // L, grp_body, 0)
                issue_in(b + 2, t)
            return carry

        lax.fori_loop(0, NBLK // 2, pair_body, 0)

        wait_in(0)
        wait_in(1)

        pltpu.sync_copy(macc, m_hbm.at[cid, :, sid])
        pltpu.sync_copy(aacc, aux_hbm.at[cid, :, sid])

    return k(x, labels)


def _tc_stats(x_tc, labels3):
    def body(x_ref, l_ref, m_ref, a_ref):
        g = pl.program_id(0)

        @pl.when(g == 0)
        def _():
            m_ref[...] = jnp.zeros_like(m_ref)
            a_ref[...] = jnp.zeros_like(a_ref)

        xb = x_ref[...]
        lab = l_ref[0, 0, :]
        cls = jax.lax.broadcasted_iota(jnp.int32, (CP, RBT), 0)
        oh = (lab[None, :] == cls)
        ohb = oh.astype(jnp.bfloat16)
        xb16 = xb.astype(jnp.bfloat16)
        m_ref[...] += jax.lax.dot_general(
            ohb, xb16, (((1,), (0,)), ((), ())),
            preferred_element_type=jnp.float32)
        x2 = jnp.sum(xb * xb, axis=1)
        rhs = jnp.stack([x2, jnp.ones_like(x2)], axis=1)
        ohf = oh.astype(jnp.float32)
        sc2 = jax.lax.dot_general(
            ohf, rhs, (((1,), (0,)), ((), ())),
            preferred_element_type=jnp.float32)
        a_ref[:, 0:2] += sc2

    return pl.pallas_call(
        body,
        grid=(F_TC // RBT,),
        out_shape=(
            jax.ShapeDtypeStruct((CP, D), jnp.float32),
            jax.ShapeDtypeStruct((CP, 128), jnp.float32),
        ),
        in_specs=[
            pl.BlockSpec((RBT, D), lambda g: (g, 0)),
            pl.BlockSpec((1, 1, RBT), lambda g: (g, 0, 0)),
        ],
        out_specs=(
            pl.BlockSpec((CP, D), lambda g: (0, 0)),
            pl.BlockSpec((CP, 128), lambda g: (0, 0)),
        ),
    )(x_tc, labels3)


CB = 200


def _finish(m_parts, aux_parts, m_tc, aux_tc, centers):
    def body(m_ref, a_ref, mt_ref, at_ref, c_ref, o_ref):
        g = pl.program_id(0)
        cen = c_ref[...]
        m = (m_ref[0] + m_ref[1]).reshape(CB, D) + mt_ref[...]
        aux = a_ref[0] + a_ref[1]
        cross = jnp.sum(m * cen, axis=1)
        s = jnp.sum(aux[:, :, 0:L], axis=(1, 2)) + at_ref[:, 0]
        cnt = jnp.sum(aux[:, :, L], axis=1) / NS + at_ref[:, 1]
        cn2 = jnp.sum(cen * cen, axis=1)
        pc = s - 2.0 * cross + cnt * cn2
        mask = pc > 0.0
        norms = jnp.where(mask, jnp.sqrt(jnp.where(mask, pc, 1.0)), 0.0)
        part = jnp.sum(norms) / C

        @pl.when(g == 0)
        def _():
            o_ref[0, 0] = 0.0

        o_ref[0, 0] += part

    return pl.pallas_call(
        body,
        grid=(C // CB,),
        out_shape=jax.ShapeDtypeStruct((1, 1), jnp.float32),
        in_specs=[
            pl.BlockSpec((NC, CB, NS, COLS), lambda g: (0, g, 0, 0)),
            pl.BlockSpec((NC, CB, NS, AW), lambda g: (0, g, 0, 0)),
            pl.BlockSpec((CB, D), lambda g: (g, 0)),
            pl.BlockSpec((CB, 128), lambda g: (g, 0)),
            pl.BlockSpec((CB, D), lambda g: (g, 0)),
        ],
        out_specs=pl.BlockSpec((1, 1), lambda g: (0, 0),
                               memory_space=pltpu.SMEM),
    )(m_parts, aux_parts, m_tc, aux_tc, centers)[0, 0]


def kernel(x, labels, centers):
    m_parts, aux_parts = _sc_stats(x[F_TC:], labels[F_TC:])
    labels3 = labels.reshape(N // RBT, 1, RBT)
    m_tc, aux_tc = _tc_stats(x, labels3)
    return _finish(m_parts, aux_parts, m_tc, aux_tc, centers)


# speedup vs baseline: 2.0111x; 1.0396x over previous
import functools

import jax
import jax.numpy as jnp
from jax import lax
from jax.experimental import pallas as pl
from jax.experimental.pallas import tpu as pltpu
from jax.experimental.pallas import tpu_sc as plsc

C = 1000
CP = 1024
N = 16384
D = 1024
NC = 2
NS = 16
L = 16
COLS = D // NS
F_TC = 14336
ROWS_PER_CORE = (N - F_TC) // NC
RB = 128
NBLK = ROWS_PER_CORE // RB
AW = 32
RBT = 512


def _sc_stats(x, labels):
    mesh = plsc.VectorSubcoreMesh(
        core_axis_name="c", subcore_axis_name="s",
        num_cores=NC, num_subcores=NS)

    @functools.partial(
        pl.kernel,
        out_type=(
            jax.ShapeDtypeStruct((NC, CP, NS, COLS), jnp.float32),
            jax.ShapeDtypeStruct((NC, CP, NS, AW), jnp.float32),
        ),
        mesh=mesh,
        scratch_types=[
            pltpu.VMEM((CP, COLS), jnp.float32),
            pltpu.VMEM((CP, AW), jnp.float32),
            pltpu.VMEM((RB, COLS), jnp.float32),
            pltpu.VMEM((RB, COLS), jnp.float32),
            pltpu.VMEM((RB,), jnp.int32),
            pltpu.VMEM((RB,), jnp.int32),
            pltpu.SemaphoreType.DMA,
            pltpu.SemaphoreType.DMA,
        ],
        compiler_params=pltpu.CompilerParams(use_tc_tiling_on_sc=False),
    )
    def k(x_hbm, lbl_hbm, m_hbm, aux_hbm,
          macc, aacc, xb0, xb1, lb0, lb1, si0, si1):
        cid = lax.axis_index("c")
        sid = lax.axis_index("s")
        row0 = cid * ROWS_PER_CORE
        col0 = sid * COLS

        zero = jnp.zeros((L,), jnp.float32)
        iota = lax.iota(jnp.int32, L)
        onev = jnp.where(iota == 0, 1.0, 0.0).astype(jnp.float32)

        def zero_body(i, carry):
            for j in range(COLS // L):
                macc[i, pl.ds(j * L, L)] = zero
            for j in range(AW // L):
                aacc[i, pl.ds(j * L, L)] = zero
            return carry

        lax.fori_loop(0, CP, zero_body, 0)

        xbs, lbs, sis = (xb0, xb1), (lb0, lb1), (si0, si1)

        def issue_in(b, p):
            r0 = row0 + (b % NBLK) * RB
            pltpu.async_copy(
                x_hbm.at[pl.ds(r0, RB), pl.ds(col0, COLS)], xbs[p], sis[p])
            pltpu.async_copy(lbl_hbm.at[pl.ds(r0, RB)], lbs[p], sis[p])

        def wait_in(p):
            pltpu.make_async_copy(
                x_hbm.at[pl.ds(row0, RB), pl.ds(col0, COLS)],
                xbs[p], sis[p]).wait()
            pltpu.make_async_copy(
                lbl_hbm.at[pl.ds(row0, RB)], lbs[p], sis[p]).wait()

        issue_in(0, 0)
        issue_in(1, 1)

        def pair_body(i, carry):
            for t in range(2):
                b = 2 * i + t
                wait_in(t)
                xb, lb = xbs[t], lbs[t]

                def grp_body(g, carry2):
                    lblv = lb[pl.ds(g * L, L)]
                    for j in range(L):
                        lab = lblv[j]
                        r = g * L + j
                        v0 = xb[r, pl.ds(0, L)]
                        v1 = xb[r, pl.ds(L, L)]
                        v2 = xb[r, pl.ds(2 * L, L)]
                        v3 = xb[r, pl.ds(3 * L, L)]
                        plsc.addupdate(macc.at[lab, pl.ds(0, L)], v0)
                        plsc.addupdate(macc.at[lab, pl.ds(L, L)], v1)
                        plsc.addupdate(macc.at[lab, pl.ds(2 * L, L)], v2)
                        plsc.addupdate(macc.at[lab, pl.ds(3 * L, L)], v3)
                        sq = ((v0 * v0 + v1 * v1) + (v2 * v2 + v3 * v3))
                        plsc.addupdate(aacc.at[lab, pl.ds(0, L)], sq)
                        plsc.addupdate(aacc.at[lab, pl.ds(L, L)], onev)
                    return carry2

                lax.fori_loop(0, RB // L, grp_body, 0)
                issue_in(b + 2, t)
            return carry

        lax.fori_loop(0, NBLK // 2, pair_body, 0)

        wait_in(0)
        wait_in(1)

        pltpu.sync_copy(macc, m_hbm.at[cid, :, sid])
        pltpu.sync_copy(aacc, aux_hbm.at[cid, :, sid])

    return k(x, labels)


def _tc_stats(x_tc, labels3):
    def body(x_ref, l_ref, m_ref, a_ref):
        g = pl.program_id(0)

        @pl.when(g == 0)
        def _():
            m_ref[...] = jnp.zeros_like(m_ref)
            a_ref[...] = jnp.zeros_like(a_ref)

        xb = x_ref[...]
        lab = l_ref[0, 0, :]
        cls = jax.lax.broadcasted_iota(jnp.int32, (CP, RBT), 0)
        oh = (lab[None, :] == cls)
        ohb = oh.astype(jnp.bfloat16)
        xb16 = xb.astype(jnp.bfloat16)
        m_ref[...] += jax.lax.dot_general(
            ohb, xb16, (((1,), (0,)), ((), ())),
            preferred_element_type=jnp.float32)
        x2 = jnp.sum(xb * xb, axis=1)
        rhs = jnp.stack([x2, jnp.ones_like(x2)], axis=1)
        ohf = oh.astype(jnp.float32)
        sc2 = jax.lax.dot_general(
            ohf, rhs, (((1,), (0,)), ((), ())),
            preferred_element_type=jnp.float32)
        a_ref[:, 0:2] += sc2

    return pl.pallas_call(
        body,
        grid=(F_TC // RBT,),
        out_shape=(
            jax.ShapeDtypeStruct((CP, D), jnp.float32),
            jax.ShapeDtypeStruct((CP, 128), jnp.float32),
        ),
        in_specs=[
            pl.BlockSpec((RBT, D), lambda g: (g, 0)),
            pl.BlockSpec((1, 1, RBT), lambda g: (g, 0, 0)),
        ],
        out_specs=(
            pl.BlockSpec((CP, D), lambda g: (0, 0)),
            pl.BlockSpec((CP, 128), lambda g: (0, 0)),
        ),
    )(x_tc, labels3)


CB = 200


def _finish(m_parts, aux_parts, m_tc, aux_tc, centers):
    def body(m_ref, a_ref, mt_ref, at_ref, c_ref, o_ref):
        g = pl.program_id(0)
        cen = c_ref[...]
        m = (m_ref[0] + m_ref[1]).reshape(CB, D) + mt_ref[...]
        aux = a_ref[0] + a_ref[1]
        cross = jnp.sum(m * cen, axis=1)
        s = jnp.sum(aux[:, :, 0:L], axis=(1, 2)) + at_ref[:, 0]
        cnt = jnp.sum(aux[:, :, L], axis=1) / NS + at_ref[:, 1]
        cn2 = jnp.sum(cen * cen, axis=1)
        pc = s - 2.0 * cross + cnt * cn2
        mask = pc > 0.0
        norms = jnp.where(mask, jnp.sqrt(jnp.where(mask, pc, 1.0)), 0.0)
        part = jnp.sum(norms) / C

        @pl.when(g == 0)
        def _():
            o_ref[0, 0] = 0.0

        o_ref[0, 0] += part

    return pl.pallas_call(
        body,
        grid=(C // CB,),
        out_shape=jax.ShapeDtypeStruct((1, 1), jnp.float32),
        in_specs=[
            pl.BlockSpec((NC, CB, NS, COLS), lambda g: (0, g, 0, 0)),
            pl.BlockSpec((NC, CB, NS, AW), lambda g: (0, g, 0, 0)),
            pl.BlockSpec((CB, D), lambda g: (g, 0)),
            pl.BlockSpec((CB, 128), lambda g: (g, 0)),
            pl.BlockSpec((CB, D), lambda g: (g, 0)),
        ],
        out_specs=pl.BlockSpec((1, 1), lambda g: (0, 0),
                               memory_space=pltpu.SMEM),
    )(m_parts, aux_parts, m_tc, aux_tc, centers)[0, 0]


def kernel(x, labels, centers):
    m_parts, aux_parts = _sc_stats(x[F_TC:], labels[F_TC:])
    labels3 = labels.reshape(N // RBT, 1, RBT)
    m_tc, aux_tc = _tc_stats(x, labels3)
    return _finish(m_parts, aux_parts, m_tc, aux_tc, centers)
